# Initial kernel scaffold; baseline (speedup 1.0000x reference)
import jax
import jax.numpy as jnp
from jax.experimental import pallas as pl


def kernel(shot_emb, mask, W, b, pos_table, mask_table, gamma, beta):
    raise NotImplementedError("write your pallas kernel here")


# speedup vs baseline: 2.5555x; 2.5555x over previous
import functools

import jax
import jax.numpy as jnp
from jax.experimental import pallas as pl

B, S, D, H = 256, 16, 2048, 768
NN = S + 1
BB = 32


def _fused_kernel(shot_ref, mask_ref, w_ref, b_ref, pos_ref, mt_ref,
                  gamma_ref, beta_ref, out_ref):
    mask = mask_ref[...]
    shot = shot_ref[...]
    mt1 = mt_ref[1:2, :].reshape(1, 1, D)
    x = jnp.where(mask[:, :, None] != 0, mt1, shot)

    x16 = x.reshape(BB * S, D).astype(jnp.bfloat16)
    w16 = w_ref[...].astype(jnp.bfloat16)
    y = jax.lax.dot_general(
        x16, w16, (((1,), (1,)), ((), ())),
        preferred_element_type=jnp.float32)
    y = y.reshape(BB, S, H)

    cls = y.mean(axis=1)
    bias = b_ref[...].reshape(1, 1, H)
    pos = pos_ref[...]
    h_rows = y + bias + pos[1:, :][None, :, :]
    h_cls = cls + bias[0] + pos[0:1, :]

    gamma = gamma_ref[...].reshape(1, H)
    beta = beta_ref[...].reshape(1, H)

    def ln(h):
        mu = jnp.mean(h, axis=-1, keepdims=True)
        var = jnp.mean((h - mu) ** 2, axis=-1, keepdims=True)
        return (h - mu) * jax.lax.rsqrt(var + 1e-12)

    out_ref[:, 0, :] = ln(h_cls) * gamma + beta
    out_ref[:, 1:, :] = ln(h_rows.reshape(BB * S, H)).reshape(BB, S, H) \
        * gamma.reshape(1, 1, H) + beta.reshape(1, 1, H)


@jax.jit
def kernel(shot_emb, mask, W, b, pos_table, mask_table, gamma, beta):
    b2 = b.reshape(1, H)
    gamma2 = gamma.reshape(1, H)
    beta2 = beta.reshape(1, H)
    grid = (B // BB,)
    return pl.pallas_call(
        _fused_kernel,
        grid=grid,
        in_specs=[
            pl.BlockSpec((BB, S, D), lambda i: (i, 0, 0)),
            pl.BlockSpec((BB, S), lambda i: (i, 0)),
            pl.BlockSpec((H, D), lambda i: (0, 0)),
            pl.BlockSpec((1, H), lambda i: (0, 0)),
            pl.BlockSpec((NN, H), lambda i: (0, 0)),
            pl.BlockSpec((2, D), lambda i: (0, 0)),
            pl.BlockSpec((1, H), lambda i: (0, 0)),
            pl.BlockSpec((1, H), lambda i: (0, 0)),
        ],
        out_specs=pl.BlockSpec((BB, NN, H), lambda i: (i, 0, 0)),
        out_shape=jax.ShapeDtypeStruct((B, NN, H), jnp.float32),
    )(shot_emb, mask, W, b2, pos_table, mask_table, gamma2, beta2)


# speedup vs baseline: 2.5603x; 1.0019x over previous
import functools

import jax
import jax.numpy as jnp
from jax.experimental import pallas as pl

B, S, D, H = 256, 16, 2048, 768
NN = S + 1
BB = 64


def _fused_kernel(shot_ref, mask_ref, w_ref, b_ref, pos_ref, mt_ref,
                  gamma_ref, beta_ref, out_ref):
    mask = mask_ref[...]
    shot16 = shot_ref[...].astype(jnp.bfloat16)
    mt1 = mt_ref[1:2, :].astype(jnp.bfloat16).reshape(1, 1, D)
    x16 = jnp.where(mask[:, :, None] != 0, mt1, shot16).reshape(BB * S, D)
    w16 = w_ref[...].astype(jnp.bfloat16)
    y = jax.lax.dot_general(
        x16, w16, (((1,), (1,)), ((), ())),
        preferred_element_type=jnp.float32)
    y = y.reshape(BB, S, H)

    cls = y.mean(axis=1)
    bias = b_ref[...].reshape(1, 1, H)
    pos = pos_ref[...]
    h_rows = y + bias + pos[1:, :][None, :, :]
    h_cls = cls + bias[0] + pos[0:1, :]

    gamma = gamma_ref[...].reshape(1, H)
    beta = beta_ref[...].reshape(1, H)

    def ln(h):
        mu = jnp.mean(h, axis=-1, keepdims=True)
        var = jnp.mean((h - mu) ** 2, axis=-1, keepdims=True)
        return (h - mu) * jax.lax.rsqrt(var + 1e-12)

    out_ref[:, 0, :] = ln(h_cls) * gamma + beta
    out_ref[:, 1:, :] = ln(h_rows.reshape(BB * S, H)).reshape(BB, S, H) \
        * gamma.reshape(1, 1, H) + beta.reshape(1, 1, H)


@jax.jit
def kernel(shot_emb, mask, W, b, pos_table, mask_table, gamma, beta):
    b2 = b.reshape(1, H)
    gamma2 = gamma.reshape(1, H)
    beta2 = beta.reshape(1, H)
    grid = (B // BB,)
    return pl.pallas_call(
        _fused_kernel,
        grid=grid,
        in_specs=[
            pl.BlockSpec((BB, S, D), lambda i: (i, 0, 0)),
            pl.BlockSpec((BB, S), lambda i: (i, 0)),
            pl.BlockSpec((H, D), lambda i: (0, 0)),
            pl.BlockSpec((1, H), lambda i: (0, 0)),
            pl.BlockSpec((NN, H), lambda i: (0, 0)),
            pl.BlockSpec((2, D), lambda i: (0, 0)),
            pl.BlockSpec((1, H), lambda i: (0, 0)),
            pl.BlockSpec((1, H), lambda i: (0, 0)),
        ],
        out_specs=pl.BlockSpec((BB, NN, H), lambda i: (i, 0, 0)),
        out_shape=jax.ShapeDtypeStruct((B, NN, H), jnp.float32),
    )(shot_emb, mask, W, b2, pos_table, mask_table, gamma2, beta2)
